# baseline (speedup 1.0000x reference)
import functools

import jax
import jax.numpy as jnp
from jax import lax
from jax.experimental import pallas as pl
from jax.experimental.pallas import tpu as pltpu
from jax.experimental.pallas import tpu_sc as plsc

N = 10000
E = 320000
H = 128
G = 64
EPS = 1e-5

NC = 2
NS = 16
NP = 10240
RPT = NP // NS
CH = 128
EP = 327680
EPW = EP // (NC * NS)
NCH = EPW // CH
EPC = EP // NC

R = 2000
NB = N // R

@functools.cache
def _build_segsum():
    mesh = plsc.VectorSubcoreMesh(core_axis_name="c", subcore_axis_name="s",
                                  num_cores=NC, num_subcores=NS)

    @functools.partial(
        pl.kernel,
        out_type=jax.ShapeDtypeStruct((NC, NP, H), jnp.float32),
        mesh=mesh,
        scratch_types=[
            pltpu.VMEM((CH,), jnp.int32),
            pltpu.VMEM((CH,), jnp.int32),
            pltpu.VMEM((CH, H), jnp.float32),
            pltpu.VMEM((CH, H), jnp.float32),
            pltpu.VMEM_SHARED((NP, H), jnp.float32),
            pltpu.SemaphoreType.DMA,
            pltpu.SemaphoreType.DMA,
        ],
    )
    def _segsum_sc(src_hbm, dst_hbm, h_hbm, zeros_hbm, out_hbm,
                   idx_s, idx_d, rows0, rows1, agg_sh, sem0, sem1):
        c = lax.axis_index("c")
        s = lax.axis_index("s")
        pltpu.sync_copy(zeros_hbm, agg_sh.at[pl.ds(s * RPT, RPT)])
        plsc.subcore_barrier()
        base = (c * NS + s) * EPW

        def chunk(j, carry):
            off = base + j * CH
            pltpu.sync_copy(src_hbm.at[pl.ds(off, CH)], idx_s)
            pltpu.sync_copy(dst_hbm.at[pl.ds(off, CH)], idx_d)
            pltpu.async_copy(h_hbm.at[idx_s], rows0, sem0).wait()
            pltpu.sync_copy(rows0, agg_sh.at[idx_d], add=True)
            return carry

        lax.fori_loop(0, NCH, chunk, 0)
        plsc.subcore_barrier()
        pltpu.sync_copy(agg_sh.at[pl.ds(s * RPT, RPT)],
                        out_hbm.at[c, pl.ds(s * RPT, RPT)])

    return _segsum_sc


def _mm_body(h_ref, a0_ref, a1_ref, w1_ref, b1_ref, w2_ref, b2_ref,
             v_ref, sm_ref, sq_ref):
    t = h_ref[...] + a0_ref[0] + a1_ref[0]
    u = jnp.maximum(
        jnp.dot(t, w1_ref[...], preferred_element_type=jnp.float32)
        + b1_ref[...], 0.0)
    v = (jnp.dot(u, w2_ref[...], preferred_element_type=jnp.float32)
         + b2_ref[...])
    v_ref[...] = v

    @pl.when(pl.program_id(0) == 0)
    def _():
        sm_ref[...] = jnp.zeros_like(sm_ref)
        sq_ref[...] = jnp.zeros_like(sq_ref)

    sm_ref[...] += jnp.sum(v, axis=0, keepdims=True)
    sq_ref[...] += jnp.sum(v * v, axis=0, keepdims=True)


_mm_call = pl.pallas_call(
    _mm_body,
    grid=(NB,),
    in_specs=[
        pl.BlockSpec((R, H), lambda i: (i, 0)),
        pl.BlockSpec((1, R, H), lambda i: (0, i, 0)),
        pl.BlockSpec((1, R, H), lambda i: (1, i, 0)),
        pl.BlockSpec((H, H), lambda i: (0, 0)),
        pl.BlockSpec((1, H), lambda i: (0, 0)),
        pl.BlockSpec((H, H), lambda i: (0, 0)),
        pl.BlockSpec((1, H), lambda i: (0, 0)),
    ],
    out_specs=[
        pl.BlockSpec((R, H), lambda i: (i, 0)),
        pl.BlockSpec((1, H), lambda i: (0, 0)),
        pl.BlockSpec((1, H), lambda i: (0, 0)),
    ],
    out_shape=[
        jax.ShapeDtypeStruct((N, H), jnp.float32),
        jax.ShapeDtypeStruct((1, H), jnp.float32),
        jax.ShapeDtypeStruct((1, H), jnp.float32),
    ],
)


def _bnpool_body(v_ref, sm_ref, sq_ref, g_ref, bb_ref, bt_ref,
                 h_ref, pool_ref, cnt_ref):
    mean = sm_ref[...] / N
    var = sq_ref[...] / N - mean * mean
    scale = g_ref[...] * lax.rsqrt(var + EPS)
    hv = jnp.maximum((v_ref[...] - mean) * scale + bb_ref[...], 0.0)
    h_ref[...] = hv
    oh = (bt_ref[0] == lax.broadcasted_iota(jnp.int32, (G, R), 0)
          ).astype(jnp.float32)

    @pl.when(pl.program_id(0) == 0)
    def _():
        pool_ref[...] = jnp.zeros_like(pool_ref)
        cnt_ref[...] = jnp.zeros_like(cnt_ref)

    pool_ref[...] += jnp.dot(oh, hv, preferred_element_type=jnp.float32)
    cnt_ref[...] += jnp.broadcast_to(
        jnp.sum(oh, axis=1, keepdims=True), (G, H))


_bnpool_call = pl.pallas_call(
    _bnpool_body,
    grid=(NB,),
    in_specs=[
        pl.BlockSpec((R, H), lambda i: (i, 0)),
        pl.BlockSpec((1, H), lambda i: (0, 0)),
        pl.BlockSpec((1, H), lambda i: (0, 0)),
        pl.BlockSpec((1, H), lambda i: (0, 0)),
        pl.BlockSpec((1, H), lambda i: (0, 0)),
        pl.BlockSpec((1, 1, R), lambda i: (i, 0, 0)),
    ],
    out_specs=[
        pl.BlockSpec((R, H), lambda i: (i, 0)),
        pl.BlockSpec((G, H), lambda i: (0, 0)),
        pl.BlockSpec((G, H), lambda i: (0, 0)),
    ],
    out_shape=[
        jax.ShapeDtypeStruct((N, H), jnp.float32),
        jax.ShapeDtypeStruct((G, H), jnp.float32),
        jax.ShapeDtypeStruct((G, H), jnp.float32),
    ],
)


def _final_body(p1_ref, p2_ref, p3_ref, cnt_ref, w_ref, b_ref, out_ref):
    cnt = jnp.maximum(cnt_ref[...], 1.0)
    o = (jnp.dot(p1_ref[...] / cnt, w_ref[0],
                 preferred_element_type=jnp.float32)
         + jnp.dot(p2_ref[...] / cnt, w_ref[1],
                   preferred_element_type=jnp.float32)
         + jnp.dot(p3_ref[...] / cnt, w_ref[2],
                   preferred_element_type=jnp.float32)
         + b_ref[...])
    nrm = jnp.sqrt(jnp.sum(o * o, axis=1, keepdims=True))
    out_ref[...] = o / jnp.maximum(nrm, 1e-12)


_final_call = pl.pallas_call(
    _final_body,
    out_shape=jax.ShapeDtypeStruct((G, H), jnp.float32),
)


def _layer(h, ei, zeros, w1, b1, w2, b2, g, bb, batch3):
    agg = _build_segsum()(ei[0], ei[1], h, zeros)
    v, sm, sq = _mm_call(h, agg, agg, w1, b1.reshape(1, H),
                         w2, b2.reshape(1, H))
    return _bnpool_call(v, sm, sq, g.reshape(1, H), bb.reshape(1, H), batch3)


def kernel(x, edge_index, batch,
           conv1_W1, conv1_b1, conv1_W2, conv1_b2,
           conv2_W1, conv2_b1, conv2_W2, conv2_b2,
           conv3_W1, conv3_b1, conv3_W2, conv3_b2,
           bn1_g, bn1_b, bn2_g, bn2_b, bn3_g, bn3_b,
           lin_W, lin_b):
    npad = EP - E
    pad = jnp.stack([jnp.zeros((npad,), jnp.int32),
                     N + jax.lax.iota(jnp.int32, npad) % (NP - N)])
    ei = jnp.concatenate([edge_index.astype(jnp.int32), pad], axis=1)
    zeros = jnp.zeros((RPT, H), jnp.float32)
    batch3 = batch.astype(jnp.int32).reshape(NB, 1, R)

    h1, pool1, cnt = _layer(x, ei, zeros, conv1_W1, conv1_b1,
                            conv1_W2, conv1_b2, bn1_g, bn1_b, batch3)
    h2, pool2, _ = _layer(h1, ei, zeros, conv2_W1, conv2_b1,
                          conv2_W2, conv2_b2, bn2_g, bn2_b, batch3)
    _, pool3, _ = _layer(h2, ei, zeros, conv3_W1, conv3_b1,
                         conv3_W2, conv3_b2, bn3_g, bn3_b, batch3)

    wsplit = lin_W.reshape(3, H, H)
    return _final_call(pool1, pool2, pool3, cnt, wsplit,
                       lin_b.reshape(1, H))


# speedup vs baseline: 2.1914x; 2.1914x over previous
import functools

import jax
import jax.numpy as jnp
from jax import lax
from jax.experimental import pallas as pl
from jax.experimental.pallas import tpu as pltpu
from jax.experimental.pallas import tpu_sc as plsc

N = 10000
E = 320000
H = 128
G = 64
EPS = 1e-5

NC = 2
NS = 16
NP = 10240
RPT = NP // NS
CH = 80
EP = 322560
EPW = EP // (NC * NS)
NCH = EPW // CH
EPC = EP // NC

R = 2000
NB = N // R

@functools.cache
def _build_segsum():
    mesh = plsc.VectorSubcoreMesh(core_axis_name="c", subcore_axis_name="s",
                                  num_cores=NC, num_subcores=NS)

    @functools.partial(
        pl.kernel,
        out_type=jax.ShapeDtypeStruct((NC, NP, H), jnp.float32),
        mesh=mesh,
        scratch_types=[
            pltpu.VMEM((CH,), jnp.int32),
            pltpu.VMEM((CH,), jnp.int32),
            pltpu.VMEM((CH,), jnp.int32),
            pltpu.VMEM((CH,), jnp.int32),
            pltpu.VMEM((CH, H), jnp.float32),
            pltpu.VMEM((CH, H), jnp.float32),
            pltpu.VMEM_SHARED((NP, H), jnp.float32),
            pltpu.SemaphoreType.DMA,
            pltpu.SemaphoreType.DMA,
        ],
    )
    def _segsum_sc(src_hbm, dst_hbm, h_hbm, zeros_hbm, out_hbm,
                   is0, id0, is1, id1, rows0, rows1, agg_sh, sem0, sem1):
        c = lax.axis_index("c")
        s = lax.axis_index("s")
        pltpu.sync_copy(zeros_hbm, agg_sh.at[pl.ds(s * RPT, RPT)])
        plsc.subcore_barrier()
        base = (c * NS + s) * EPW

        pltpu.sync_copy(src_hbm.at[pl.ds(base, CH)], is0)
        pltpu.sync_copy(dst_hbm.at[pl.ds(base, CH)], id0)
        pltpu.async_copy(h_hbm.at[is0], rows0, sem0)

        def step(i, carry):
            off1 = base + (2 * i + 1) * CH
            pltpu.sync_copy(src_hbm.at[pl.ds(off1, CH)], is1)
            pltpu.sync_copy(dst_hbm.at[pl.ds(off1, CH)], id1)
            pltpu.async_copy(h_hbm.at[is1], rows1, sem1)
            pltpu.make_async_copy(h_hbm.at[is0], rows0, sem0).wait()
            pltpu.sync_copy(rows0, agg_sh.at[id0], add=True)

            @pl.when(i < NCH // 2 - 1)
            def _():
                off2 = base + (2 * i + 2) * CH
                pltpu.sync_copy(src_hbm.at[pl.ds(off2, CH)], is0)
                pltpu.sync_copy(dst_hbm.at[pl.ds(off2, CH)], id0)
                pltpu.async_copy(h_hbm.at[is0], rows0, sem0)

            pltpu.make_async_copy(h_hbm.at[is1], rows1, sem1).wait()
            pltpu.sync_copy(rows1, agg_sh.at[id1], add=True)
            return carry

        lax.fori_loop(0, NCH // 2, step, 0)
        plsc.subcore_barrier()
        pltpu.sync_copy(agg_sh.at[pl.ds(s * RPT, RPT)],
                        out_hbm.at[c, pl.ds(s * RPT, RPT)])

    return _segsum_sc


def _mm_body(h_ref, a0_ref, a1_ref, w1_ref, b1_ref, w2_ref, b2_ref,
             v_ref, sm_ref, sq_ref):
    t = h_ref[...] + a0_ref[0] + a1_ref[0]
    u = jnp.maximum(
        jnp.dot(t, w1_ref[...], preferred_element_type=jnp.float32)
        + b1_ref[...], 0.0)
    v = (jnp.dot(u, w2_ref[...], preferred_element_type=jnp.float32)
         + b2_ref[...])
    v_ref[...] = v

    @pl.when(pl.program_id(0) == 0)
    def _():
        sm_ref[...] = jnp.zeros_like(sm_ref)
        sq_ref[...] = jnp.zeros_like(sq_ref)

    sm_ref[...] += jnp.sum(v, axis=0, keepdims=True)
    sq_ref[...] += jnp.sum(v * v, axis=0, keepdims=True)


_mm_call = pl.pallas_call(
    _mm_body,
    grid=(NB,),
    in_specs=[
        pl.BlockSpec((R, H), lambda i: (i, 0)),
        pl.BlockSpec((1, R, H), lambda i: (0, i, 0)),
        pl.BlockSpec((1, R, H), lambda i: (1, i, 0)),
        pl.BlockSpec((H, H), lambda i: (0, 0)),
        pl.BlockSpec((1, H), lambda i: (0, 0)),
        pl.BlockSpec((H, H), lambda i: (0, 0)),
        pl.BlockSpec((1, H), lambda i: (0, 0)),
    ],
    out_specs=[
        pl.BlockSpec((R, H), lambda i: (i, 0)),
        pl.BlockSpec((1, H), lambda i: (0, 0)),
        pl.BlockSpec((1, H), lambda i: (0, 0)),
    ],
    out_shape=[
        jax.ShapeDtypeStruct((N, H), jnp.float32),
        jax.ShapeDtypeStruct((1, H), jnp.float32),
        jax.ShapeDtypeStruct((1, H), jnp.float32),
    ],
)


def _bnpool_body(v_ref, sm_ref, sq_ref, g_ref, bb_ref, bt_ref,
                 h_ref, pool_ref, cnt_ref):
    mean = sm_ref[...] / N
    var = sq_ref[...] / N - mean * mean
    scale = g_ref[...] * lax.rsqrt(var + EPS)
    hv = jnp.maximum((v_ref[...] - mean) * scale + bb_ref[...], 0.0)
    h_ref[...] = hv
    oh = (bt_ref[0] == lax.broadcasted_iota(jnp.int32, (G, R), 0)
          ).astype(jnp.float32)

    @pl.when(pl.program_id(0) == 0)
    def _():
        pool_ref[...] = jnp.zeros_like(pool_ref)
        cnt_ref[...] = jnp.zeros_like(cnt_ref)

    pool_ref[...] += jnp.dot(oh, hv, preferred_element_type=jnp.float32)
    cnt_ref[...] += jnp.broadcast_to(
        jnp.sum(oh, axis=1, keepdims=True), (G, H))


_bnpool_call = pl.pallas_call(
    _bnpool_body,
    grid=(NB,),
    in_specs=[
        pl.BlockSpec((R, H), lambda i: (i, 0)),
        pl.BlockSpec((1, H), lambda i: (0, 0)),
        pl.BlockSpec((1, H), lambda i: (0, 0)),
        pl.BlockSpec((1, H), lambda i: (0, 0)),
        pl.BlockSpec((1, H), lambda i: (0, 0)),
        pl.BlockSpec((1, 1, R), lambda i: (i, 0, 0)),
    ],
    out_specs=[
        pl.BlockSpec((R, H), lambda i: (i, 0)),
        pl.BlockSpec((G, H), lambda i: (0, 0)),
        pl.BlockSpec((G, H), lambda i: (0, 0)),
    ],
    out_shape=[
        jax.ShapeDtypeStruct((N, H), jnp.float32),
        jax.ShapeDtypeStruct((G, H), jnp.float32),
        jax.ShapeDtypeStruct((G, H), jnp.float32),
    ],
)


def _final_body(p1_ref, p2_ref, p3_ref, cnt_ref, w_ref, b_ref, out_ref):
    cnt = jnp.maximum(cnt_ref[...], 1.0)
    o = (jnp.dot(p1_ref[...] / cnt, w_ref[0],
                 preferred_element_type=jnp.float32)
         + jnp.dot(p2_ref[...] / cnt, w_ref[1],
                   preferred_element_type=jnp.float32)
         + jnp.dot(p3_ref[...] / cnt, w_ref[2],
                   preferred_element_type=jnp.float32)
         + b_ref[...])
    nrm = jnp.sqrt(jnp.sum(o * o, axis=1, keepdims=True))
    out_ref[...] = o / jnp.maximum(nrm, 1e-12)


_final_call = pl.pallas_call(
    _final_body,
    out_shape=jax.ShapeDtypeStruct((G, H), jnp.float32),
)


def _layer(h, ei, zeros, w1, b1, w2, b2, g, bb, batch3):
    agg = _build_segsum()(ei[0], ei[1], h, zeros)
    v, sm, sq = _mm_call(h, agg, agg, w1, b1.reshape(1, H),
                         w2, b2.reshape(1, H))
    return _bnpool_call(v, sm, sq, g.reshape(1, H), bb.reshape(1, H), batch3)


def kernel(x, edge_index, batch,
           conv1_W1, conv1_b1, conv1_W2, conv1_b2,
           conv2_W1, conv2_b1, conv2_W2, conv2_b2,
           conv3_W1, conv3_b1, conv3_W2, conv3_b2,
           bn1_g, bn1_b, bn2_g, bn2_b, bn3_g, bn3_b,
           lin_W, lin_b):
    npad = EP - E
    pad = jnp.stack([jnp.zeros((npad,), jnp.int32),
                     N + jax.lax.iota(jnp.int32, npad) % (NP - N)])
    ei = jnp.concatenate([edge_index.astype(jnp.int32), pad], axis=1)
    zeros = jnp.zeros((RPT, H), jnp.float32)
    batch3 = batch.astype(jnp.int32).reshape(NB, 1, R)

    h1, pool1, cnt = _layer(x, ei, zeros, conv1_W1, conv1_b1,
                            conv1_W2, conv1_b2, bn1_g, bn1_b, batch3)
    h2, pool2, _ = _layer(h1, ei, zeros, conv2_W1, conv2_b1,
                          conv2_W2, conv2_b2, bn2_g, bn2_b, batch3)
    _, pool3, _ = _layer(h2, ei, zeros, conv3_W1, conv3_b1,
                         conv3_W2, conv3_b2, bn3_g, bn3_b, batch3)

    wsplit = lin_W.reshape(3, H, H)
    return _final_call(pool1, pool2, pool3, cnt, wsplit,
                       lin_b.reshape(1, H))
